# baseline (speedup 1.0000x reference)
import functools

import jax
import jax.numpy as jnp
from jax import lax
from jax.experimental import pallas as pl
from jax.experimental.pallas import tpu as pltpu
from jax.experimental.pallas import tpu_sc as plsc

N = 4096
IN_DIM = 512
HID = 256
OUT_DIM = 64
PROP = 4
LAM = 1.0 / 0.9 - 1.0
Q = 0.75
A_SCAD = 3.7
EPS = 1e-8

BLK = 256
NBLK = N // BLK
CAP = 1 << 18
INF_BITS = 0x7F800000


def _prep_kernel(F_ref, W1_ref, b1_ref, W2_ref, b2_ref, A_ref,
                 F0_ref, D_ref, Xn_ref):
    h = jnp.dot(F_ref[:], W1_ref[:], preferred_element_type=jnp.float32)
    h = jnp.maximum(h + b1_ref[:], 0.0)
    f0 = jnp.dot(h, W2_ref[:], preferred_element_type=jnp.float32) + b2_ref[:]
    F0_ref[:] = f0
    d = jnp.sum(A_ref[:], axis=1) + 1.0
    D_ref[0, 0, :] = d
    Xn_ref[:] = f0 / jnp.sqrt(d)[:, None]


def _dist_kernel(Xn_blk_ref, Xn_ref, y2_ref):
    xn = Xn_ref[:]
    rows = Xn_blk_ref[:]
    s_full = jnp.sum(xn * xn, axis=1)
    s_rows = jnp.sum(rows * rows, axis=1)
    g = lax.dot_general(rows, xn, (((1,), (1,)), ((), ())),
                        preferred_element_type=jnp.float32)
    y2_ref[:] = jnp.maximum(s_rows[:, None] + s_full[None, :] - 2.0 * g, 0.0)


def _gamma_kernel(y2e_ref, D_ref, lam_ref):
    d = D_ref[:]
    m_f = jnp.sum(d) - float(N)
    h = Q * (m_f - 1.0)
    k_f = jnp.floor(h)
    frac = h - k_f
    kd = k_f.astype(jnp.int32)
    k1 = kd // 2
    k2 = (kd + 1) // 2

    bits = lax.bitcast_convert_type(y2e_ref[:], jnp.int32)

    def body(_, carry):
        lo1, hi1, lo2, hi2 = carry
        mid1 = lo1 + (hi1 - lo1) // 2
        mid2 = lo2 + (hi2 - lo2) // 2
        c1 = jnp.sum((bits <= mid1).astype(jnp.int32))
        c2 = jnp.sum((bits <= mid2).astype(jnp.int32))
        ok1 = c1 >= k1 + 1
        ok2 = c2 >= k2 + 1
        return (jnp.where(ok1, lo1, mid1 + 1), jnp.where(ok1, mid1, hi1),
                jnp.where(ok2, lo2, mid2 + 1), jnp.where(ok2, mid2, hi2))

    z = jnp.int32(0)
    inf = jnp.int32(INF_BITS)
    _, hi1, _, hi2 = lax.fori_loop(0, 31, body, (z, inf, z, inf))
    y_k = jnp.sqrt(lax.bitcast_convert_type(hi1, jnp.float32))
    y_k1 = jnp.sqrt(lax.bitcast_convert_type(hi2, jnp.float32))
    gamma = jnp.where(frac > 0.0, y_k + frac * (y_k1 - y_k), y_k)
    gamma = jnp.maximum(gamma, EPS)
    lam_ref[:, :] = jnp.reshape(gamma / A_SCAD, (1, 1))


def _prop_kernel(A_ref, Xn_blk_ref, Xn_ref, D_ref, F0_ref, lam_ref,
                 Fc_ref, Xn_out_ref):
    i = pl.program_id(0)
    lam = lam_ref[0, 0]
    xn = Xn_ref[:]
    rows = Xn_blk_ref[:]
    s_full = jnp.sum(xn * xn, axis=1)
    s_rows = jnp.sum(rows * rows, axis=1)
    g = lax.dot_general(rows, xn, (((1,), (1,)), ((), ())),
                        preferred_element_type=jnp.float32)
    y2 = jnp.maximum(s_rows[:, None] + s_full[None, :] - 2.0 * g, 0.0)
    y = jnp.sqrt(y2)
    alam = A_SCAD * lam
    mid = (alam - y) / ((A_SCAD - 1.0) * y + 1e-12)
    w = jnp.where(y <= lam, 1.0, jnp.where(y <= alam, mid, 0.0))
    s = w * A_ref[:]
    d_rows = D_ref[0, pl.ds(i * BLK, BLK)]
    qhat = jnp.sum(s, axis=1) / d_rows + LAM
    agg = jnp.dot(s, Xn_ref[:], preferred_element_type=jnp.float32)
    ds_rows = jnp.sqrt(d_rows)
    fc = (agg / ds_rows[:, None] + LAM * F0_ref[:]) / qhat[:, None]
    Fc_ref[:] = fc
    Xn_out_ref[:] = fc / ds_rows[:, None]


_NC = 2
_NS = 16
_NW = _NC * _NS
_TROWS = N // _NW
_NSUB = 16
_SROWS = _TROWS // _NSUB
_ECAP = 640
_EPAD = _ECAP + 16
_TCAP = _NSUB * _ECAP
_TOTCAP = _NW * _TCAP

def _wid():
    return lax.axis_index("s") * _NC + lax.axis_index("c")


@functools.cache
def _sc_kernels():
    mesh = plsc.VectorSubcoreMesh(core_axis_name="c", subcore_axis_name="s")

    @functools.partial(
        pl.kernel,
        out_type=jax.ShapeDtypeStruct((_TOTCAP,), jnp.int32),
        mesh=mesh,
        compiler_params=pltpu.CompilerParams(needs_layout_passes=False),
        scratch_types=[
            pltpu.VMEM((_SROWS, N), jnp.float32),
            pltpu.VMEM((_EPAD,), jnp.int32),
        ],
    )
    def extract(a_hbm, eidx_hbm, slab_v, out_v):
        wid = _wid()
        row0 = wid * _TROWS
        iota = lax.iota(jnp.int32, 16)
        neg1 = jnp.full((16,), -1, jnp.int32)

        def sub(s, _):
            pltpu.sync_copy(a_hbm.at[pl.ds(row0 + s * _SROWS, _SROWS)], slab_v)
            for j in range(_EPAD // 16):
                out_v[pl.ds(j * 16, 16)] = neg1
            row_base = row0 + s * _SROWS

            def grp(g, off):
                cols = g * 16 + iota
                for r in range(_SROWS):
                    v = slab_v[r, pl.ds(g * 16, 16)]
                    m = (v > 0.5) & (cols > row_base + r)
                    plsc.store_compressed(out_v.at[pl.ds(off, 16)],
                                          r * N + g * 16 + iota, mask=m)
                    pcnt = plsc.all_reduce_population_count(m)[0]
                    off = jnp.minimum(off + pcnt, _ECAP)
                return off

            lax.fori_loop(row_base // 16, N // 16, grp, jnp.int32(0))
            pltpu.sync_copy(out_v.at[pl.ds(0, _ECAP)],
                            eidx_hbm.at[pl.ds(wid * _TCAP + s * _ECAP, _ECAP)])
            return _

        lax.fori_loop(0, _NSUB, sub, None)

    @functools.partial(
        pl.kernel,
        out_type=jax.ShapeDtypeStruct((_TOTCAP,), jnp.float32),
        mesh=mesh,
        compiler_params=pltpu.CompilerParams(needs_layout_passes=False),
        scratch_types=[
            pltpu.VMEM((_SROWS, N), jnp.float32),
            pltpu.VMEM((_TCAP,), jnp.int32),
            pltpu.VMEM((_TCAP,), jnp.float32),
        ],
    )
    def gather(y2_hbm, eidx_hbm, out_hbm, slab_v, eidx_v, out_v):
        wid = _wid()
        row0 = wid * _TROWS
        pltpu.sync_copy(eidx_hbm.at[pl.ds(wid * _TCAP, _TCAP)], eidx_v)
        inf = jnp.full((16,), jnp.inf, jnp.float32)

        def sub(s, _):
            pltpu.sync_copy(y2_hbm.at[pl.ds(row0 + s * _SROWS, _SROWS)],
                            slab_v)

            def grp(g, carry):
                o = s * _ECAP + g * 16
                idx = eidx_v[pl.ds(o, 16)]
                m = idx >= 0
                ic = jnp.maximum(idx, 0)
                vals = plsc.load_gather(
                    slab_v, [lax.shift_right_logical(ic, 12), ic & (N - 1)])
                out_v[pl.ds(o, 16)] = jnp.where(m, vals, inf)
                return carry

            lax.fori_loop(0, _ECAP // 16, grp, None)
            return _

        lax.fori_loop(0, _NSUB, sub, None)
        pltpu.sync_copy(out_v, out_hbm.at[pl.ds(wid * _TCAP, _TCAP)])

    return extract, gather


def _prep_call(F, W1, b1, W2, b2, A):
    return pl.pallas_call(
        _prep_kernel,
        grid=(NBLK,),
        in_specs=[
            pl.BlockSpec((BLK, IN_DIM), lambda i: (i, 0)),
            pl.BlockSpec((IN_DIM, HID), lambda i: (0, 0)),
            pl.BlockSpec((1, HID), lambda i: (0, 0)),
            pl.BlockSpec((HID, OUT_DIM), lambda i: (0, 0)),
            pl.BlockSpec((1, OUT_DIM), lambda i: (0, 0)),
            pl.BlockSpec((BLK, N), lambda i: (i, 0)),
        ],
        out_specs=[
            pl.BlockSpec((BLK, OUT_DIM), lambda i: (i, 0)),
            pl.BlockSpec((1, 1, BLK), lambda i: (i, 0, 0)),
            pl.BlockSpec((BLK, OUT_DIM), lambda i: (i, 0)),
        ],
        out_shape=[
            jax.ShapeDtypeStruct((N, OUT_DIM), jnp.float32),
            jax.ShapeDtypeStruct((NBLK, 1, BLK), jnp.float32),
            jax.ShapeDtypeStruct((N, OUT_DIM), jnp.float32),
        ],
    )(F, W1, b1, W2, b2, A)


def _dist_call(Xn):
    return pl.pallas_call(
        _dist_kernel,
        grid=(NBLK,),
        in_specs=[
            pl.BlockSpec((BLK, OUT_DIM), lambda i: (i, 0)),
            pl.BlockSpec((N, OUT_DIM), lambda i: (0, 0)),
        ],
        out_specs=pl.BlockSpec((BLK, N), lambda i: (i, 0)),
        out_shape=jax.ShapeDtypeStruct((N, N), jnp.float32),
    )(Xn, Xn)


def _gamma_call(y2e, D):
    return pl.pallas_call(
        _gamma_kernel,
        in_specs=[
            pl.BlockSpec((_TOTCAP // 128, 128), lambda: (0, 0)),
            pl.BlockSpec((1, N), lambda: (0, 0)),
        ],
        out_specs=pl.BlockSpec((1, 1), lambda: (0, 0)),
        out_shape=jax.ShapeDtypeStruct((1, 1), jnp.float32),
    )(y2e, D)


def _prop_call(A, Xn, D, F0, lam):
    return pl.pallas_call(
        _prop_kernel,
        grid=(NBLK,),
        in_specs=[
            pl.BlockSpec((BLK, N), lambda i: (i, 0)),
            pl.BlockSpec((BLK, OUT_DIM), lambda i: (i, 0)),
            pl.BlockSpec((N, OUT_DIM), lambda i: (0, 0)),
            pl.BlockSpec((1, N), lambda i: (0, 0)),
            pl.BlockSpec((BLK, OUT_DIM), lambda i: (i, 0)),
            pl.BlockSpec((1, 1), lambda i: (0, 0)),
        ],
        out_specs=[
            pl.BlockSpec((BLK, OUT_DIM), lambda i: (i, 0)),
            pl.BlockSpec((BLK, OUT_DIM), lambda i: (i, 0)),
        ],
        out_shape=[
            jax.ShapeDtypeStruct((N, OUT_DIM), jnp.float32),
            jax.ShapeDtypeStruct((N, OUT_DIM), jnp.float32),
        ],
    )(A, Xn, Xn, D, F0, lam)


def kernel(A, F, W1, b1, W2, b2):
    _extract, _gather = _sc_kernels()
    eidx = _extract(A)
    F0, D3, Xn = _prep_call(F, W1, b1.reshape(1, HID), W2,
                            b2.reshape(1, OUT_DIM), A)
    D = D3.reshape(1, N)

    Fc = F0
    for _ in range(PROP):
        y2 = _dist_call(Xn)
        y2e = _gather(y2, eidx)
        lam = _gamma_call(y2e.reshape(_TOTCAP // 128, 128), D)
        Fc, Xn = _prop_call(A, Xn, D, F0, lam)
    return Fc


# speedup vs baseline: 1.0323x; 1.0323x over previous
import functools

import jax
import jax.numpy as jnp
from jax import lax
from jax.experimental import pallas as pl
from jax.experimental.pallas import tpu as pltpu
from jax.experimental.pallas import tpu_sc as plsc

N = 4096
IN_DIM = 512
HID = 256
OUT_DIM = 64
PROP = 4
LAM = 1.0 / 0.9 - 1.0
Q = 0.75
A_SCAD = 3.7
EPS = 1e-8

BLK = 256
NBLK = N // BLK
CAP = 1 << 18
INF_BITS = 0x7F800000


def _prep_kernel(F_ref, W1_ref, b1_ref, W2_ref, b2_ref, A_ref,
                 F0_ref, D_ref, Xn_ref):
    h = jnp.dot(F_ref[:], W1_ref[:], preferred_element_type=jnp.float32)
    h = jnp.maximum(h + b1_ref[:], 0.0)
    f0 = jnp.dot(h, W2_ref[:], preferred_element_type=jnp.float32) + b2_ref[:]
    F0_ref[:] = f0
    d = jnp.sum(A_ref[:], axis=1) + 1.0
    D_ref[0, 0, :] = d
    Xn_ref[:] = f0 / jnp.sqrt(d)[:, None]


def _dist_kernel(Xn_blk_ref, Xn_ref, y2_ref):
    xn = Xn_ref[:]
    rows = Xn_blk_ref[:]
    s_full = jnp.sum(xn * xn, axis=1)
    s_rows = jnp.sum(rows * rows, axis=1)
    g = lax.dot_general(rows, xn, (((1,), (1,)), ((), ())),
                        preferred_element_type=jnp.float32)
    y2_ref[:] = jnp.maximum(s_rows[:, None] + s_full[None, :] - 2.0 * g, 0.0)


def _gamma_kernel(y2e_ref, D_ref, lam_ref):
    d = D_ref[:]
    m_f = jnp.sum(d) - float(N)
    h = Q * (m_f - 1.0)
    k_f = jnp.floor(h)
    frac = h - k_f
    kd = k_f.astype(jnp.int32)
    k1 = kd
    k2 = kd + 1

    bits = lax.bitcast_convert_type(y2e_ref[:], jnp.int32)

    def body(_, carry):
        lo1, hi1, lo2, hi2 = carry
        mid1 = lo1 + (hi1 - lo1) // 2
        mid2 = lo2 + (hi2 - lo2) // 2
        c1 = jnp.sum((bits <= mid1).astype(jnp.int32))
        c2 = jnp.sum((bits <= mid2).astype(jnp.int32))
        ok1 = c1 >= k1 + 1
        ok2 = c2 >= k2 + 1
        return (jnp.where(ok1, lo1, mid1 + 1), jnp.where(ok1, mid1, hi1),
                jnp.where(ok2, lo2, mid2 + 1), jnp.where(ok2, mid2, hi2))

    z = jnp.int32(0)
    inf = jnp.int32(INF_BITS)
    _, hi1, _, hi2 = lax.fori_loop(0, 31, body, (z, inf, z, inf))
    y_k = jnp.sqrt(lax.bitcast_convert_type(hi1, jnp.float32))
    y_k1 = jnp.sqrt(lax.bitcast_convert_type(hi2, jnp.float32))
    gamma = jnp.where(frac > 0.0, y_k + frac * (y_k1 - y_k), y_k)
    gamma = jnp.maximum(gamma, EPS)
    lam_ref[:, :] = jnp.reshape(gamma / A_SCAD, (1, 1))


def _prop_kernel(A_ref, Xn_blk_ref, Xn_ref, D_ref, F0_ref, lam_ref,
                 Fc_ref, Xn_out_ref):
    i = pl.program_id(0)
    lam = lam_ref[0, 0]
    xn = Xn_ref[:]
    rows = Xn_blk_ref[:]
    s_full = jnp.sum(xn * xn, axis=1)
    s_rows = jnp.sum(rows * rows, axis=1)
    g = lax.dot_general(rows, xn, (((1,), (1,)), ((), ())),
                        preferred_element_type=jnp.float32)
    y2 = jnp.maximum(s_rows[:, None] + s_full[None, :] - 2.0 * g, 0.0)
    y = jnp.sqrt(y2)
    alam = A_SCAD * lam
    mid = (alam - y) / ((A_SCAD - 1.0) * y + 1e-12)
    w = jnp.where(y <= lam, 1.0, jnp.where(y <= alam, mid, 0.0))
    s = w * A_ref[:]
    d_rows = D_ref[0, pl.ds(i * BLK, BLK)]
    qhat = jnp.sum(s, axis=1) / d_rows + LAM
    agg = jnp.dot(s, Xn_ref[:], preferred_element_type=jnp.float32)
    ds_rows = jnp.sqrt(d_rows)
    fc = (agg / ds_rows[:, None] + LAM * F0_ref[:]) / qhat[:, None]
    Fc_ref[:] = fc
    Xn_out_ref[:] = fc / ds_rows[:, None]


_NC = 2
_NS = 16
_NW = _NC * _NS
_TROWS = N // _NW
_NSUB = 16
_SROWS = _TROWS // _NSUB
_ECAP = 640
_EPAD = _ECAP + 16
_TCAP = _NSUB * _ECAP
_TOTCAP = _NW * _TCAP

def _wid():
    return lax.axis_index("s") * _NC + lax.axis_index("c")


@functools.cache
def _sc_kernels():
    mesh = plsc.VectorSubcoreMesh(core_axis_name="c", subcore_axis_name="s")

    @functools.partial(
        pl.kernel,
        out_type=jax.ShapeDtypeStruct((_TOTCAP,), jnp.int32),
        mesh=mesh,
        compiler_params=pltpu.CompilerParams(needs_layout_passes=False),
        scratch_types=[
            pltpu.VMEM((_SROWS, N), jnp.float32),
            pltpu.VMEM((_EPAD,), jnp.int32),
        ],
    )
    def extract(a_hbm, eidx_hbm, slab_v, out_v):
        wid = _wid()
        row0 = wid * _TROWS
        iota = lax.iota(jnp.int32, 16)
        neg1 = jnp.full((16,), -1, jnp.int32)

        def sub(s, _):
            pltpu.sync_copy(a_hbm.at[pl.ds(row0 + s * _SROWS, _SROWS)], slab_v)
            for j in range(_EPAD // 16):
                out_v[pl.ds(j * 16, 16)] = neg1

            def grp(g, off):
                for r in range(_SROWS):
                    v = slab_v[r, pl.ds(g * 16, 16)]
                    m = v > 0.5
                    plsc.store_compressed(out_v.at[pl.ds(off, 16)],
                                          r * N + g * 16 + iota, mask=m)
                    pcnt = plsc.all_reduce_population_count(m)[0]
                    off = jnp.minimum(off + pcnt, _ECAP)
                return off

            lax.fori_loop(0, N // 16, grp, jnp.int32(0))
            pltpu.sync_copy(out_v.at[pl.ds(0, _ECAP)],
                            eidx_hbm.at[pl.ds(wid * _TCAP + s * _ECAP, _ECAP)])
            return _

        lax.fori_loop(0, _NSUB, sub, None)

    @functools.partial(
        pl.kernel,
        out_type=jax.ShapeDtypeStruct((_TOTCAP,), jnp.float32),
        mesh=mesh,
        compiler_params=pltpu.CompilerParams(needs_layout_passes=False),
        scratch_types=[
            pltpu.VMEM((_SROWS, N), jnp.float32),
            pltpu.VMEM((_TCAP,), jnp.int32),
            pltpu.VMEM((_TCAP,), jnp.float32),
        ],
    )
    def gather(y2_hbm, eidx_hbm, out_hbm, slab_v, eidx_v, out_v):
        wid = _wid()
        row0 = wid * _TROWS
        pltpu.sync_copy(eidx_hbm.at[pl.ds(wid * _TCAP, _TCAP)], eidx_v)
        inf = jnp.full((16,), jnp.inf, jnp.float32)

        def sub(s, _):
            pltpu.sync_copy(y2_hbm.at[pl.ds(row0 + s * _SROWS, _SROWS)],
                            slab_v)

            def grp(g, carry):
                o = s * _ECAP + g * 16
                idx = eidx_v[pl.ds(o, 16)]
                m = idx >= 0
                ic = jnp.maximum(idx, 0)
                vals = plsc.load_gather(
                    slab_v, [lax.shift_right_logical(ic, 12), ic & (N - 1)])
                out_v[pl.ds(o, 16)] = jnp.where(m, vals, inf)
                return carry

            lax.fori_loop(0, _ECAP // 16, grp, None)
            return _

        lax.fori_loop(0, _NSUB, sub, None)
        pltpu.sync_copy(out_v, out_hbm.at[pl.ds(wid * _TCAP, _TCAP)])

    return extract, gather


def _prep_call(F, W1, b1, W2, b2, A):
    return pl.pallas_call(
        _prep_kernel,
        grid=(NBLK,),
        in_specs=[
            pl.BlockSpec((BLK, IN_DIM), lambda i: (i, 0)),
            pl.BlockSpec((IN_DIM, HID), lambda i: (0, 0)),
            pl.BlockSpec((1, HID), lambda i: (0, 0)),
            pl.BlockSpec((HID, OUT_DIM), lambda i: (0, 0)),
            pl.BlockSpec((1, OUT_DIM), lambda i: (0, 0)),
            pl.BlockSpec((BLK, N), lambda i: (i, 0)),
        ],
        out_specs=[
            pl.BlockSpec((BLK, OUT_DIM), lambda i: (i, 0)),
            pl.BlockSpec((1, 1, BLK), lambda i: (i, 0, 0)),
            pl.BlockSpec((BLK, OUT_DIM), lambda i: (i, 0)),
        ],
        out_shape=[
            jax.ShapeDtypeStruct((N, OUT_DIM), jnp.float32),
            jax.ShapeDtypeStruct((NBLK, 1, BLK), jnp.float32),
            jax.ShapeDtypeStruct((N, OUT_DIM), jnp.float32),
        ],
    )(F, W1, b1, W2, b2, A)


def _dist_call(Xn):
    return pl.pallas_call(
        _dist_kernel,
        grid=(NBLK,),
        in_specs=[
            pl.BlockSpec((BLK, OUT_DIM), lambda i: (i, 0)),
            pl.BlockSpec((N, OUT_DIM), lambda i: (0, 0)),
        ],
        out_specs=pl.BlockSpec((BLK, N), lambda i: (i, 0)),
        out_shape=jax.ShapeDtypeStruct((N, N), jnp.float32),
    )(Xn, Xn)


def _gamma_call(y2e, D):
    return pl.pallas_call(
        _gamma_kernel,
        in_specs=[
            pl.BlockSpec((_TOTCAP // 128, 128), lambda: (0, 0)),
            pl.BlockSpec((1, N), lambda: (0, 0)),
        ],
        out_specs=pl.BlockSpec((1, 1), lambda: (0, 0)),
        out_shape=jax.ShapeDtypeStruct((1, 1), jnp.float32),
    )(y2e, D)


def _prop_call(A, Xn, D, F0, lam):
    return pl.pallas_call(
        _prop_kernel,
        grid=(NBLK,),
        in_specs=[
            pl.BlockSpec((BLK, N), lambda i: (i, 0)),
            pl.BlockSpec((BLK, OUT_DIM), lambda i: (i, 0)),
            pl.BlockSpec((N, OUT_DIM), lambda i: (0, 0)),
            pl.BlockSpec((1, N), lambda i: (0, 0)),
            pl.BlockSpec((BLK, OUT_DIM), lambda i: (i, 0)),
            pl.BlockSpec((1, 1), lambda i: (0, 0)),
        ],
        out_specs=[
            pl.BlockSpec((BLK, OUT_DIM), lambda i: (i, 0)),
            pl.BlockSpec((BLK, OUT_DIM), lambda i: (i, 0)),
        ],
        out_shape=[
            jax.ShapeDtypeStruct((N, OUT_DIM), jnp.float32),
            jax.ShapeDtypeStruct((N, OUT_DIM), jnp.float32),
        ],
    )(A, Xn, Xn, D, F0, lam)


def kernel(A, F, W1, b1, W2, b2):
    _extract, _gather = _sc_kernels()
    eidx = _extract(A)
    F0, D3, Xn = _prep_call(F, W1, b1.reshape(1, HID), W2,
                            b2.reshape(1, OUT_DIM), A)
    D = D3.reshape(1, N)

    Fc = F0
    for _ in range(PROP):
        y2 = _dist_call(Xn)
        y2e = _gather(y2, eidx)
        lam = _gamma_call(y2e.reshape(_TOTCAP // 128, 128), D)
        Fc, Xn = _prop_call(A, Xn, D, F0, lam)
    return Fc
